# Initial kernel scaffold; baseline (speedup 1.0000x reference)
import jax
import jax.numpy as jnp
from jax.experimental import pallas as pl


def kernel(features, edge_index, edge_weight, self_weight, W1, b1, W2, b2):
    raise NotImplementedError("write your pallas kernel here")


# speedup vs baseline: 9.5571x; 9.5571x over previous
import functools

import jax
import jax.numpy as jnp
from jax import lax
from jax.experimental import pallas as pl
from jax.experimental.pallas import tpu as pltpu
from jax.experimental.pallas import tpu_sc as plsc

NC = 2
NS = 16
NW = NC * NS
K = 128


def _matmul_tc(x, w):
  def body(x_ref, w_ref, o_ref):
    o_ref[...] = lax.dot_general(
        x_ref[...], w_ref[...], (((1,), (1,)), ((), ())),
        preferred_element_type=jnp.float32)
  return pl.pallas_call(
      body,
      out_shape=jax.ShapeDtypeStruct((x.shape[0], w.shape[0]), jnp.float32),
  )(x, w)


def _combine_relu_tc(p, sw, agg0, agg1, b):
  def body(p_ref, sw_ref, a0_ref, a1_ref, b_ref, o_ref):
    o_ref[...] = jnp.maximum(
        p_ref[...] * (sw_ref[...] + 1.0) + a0_ref[...] + a1_ref[...]
        + b_ref[...], 0.0)
  return pl.pallas_call(
      body,
      out_shape=jax.ShapeDtypeStruct(p.shape, jnp.float32),
  )(p, sw, agg0, agg1, b.reshape(1, -1))


def _final_tc(x, sw, agg0, agg1, w2, b2):
  def body(x_ref, sw_ref, a0_ref, a1_ref, w_ref, b_ref, o_ref):
    t = x_ref[...] * (sw_ref[...] + 1.0) + a0_ref[...] + a1_ref[...]
    o_ref[...] = lax.dot_general(
        t, w_ref[...], (((1,), (1,)), ((), ())),
        preferred_element_type=jnp.float32) + b_ref[...]
  return pl.pallas_call(
      body,
      out_shape=jax.ShapeDtypeStruct((x.shape[0], w2.shape[0]), jnp.float32),
  )(x, sw, agg0, agg1, w2, b2.reshape(1, -1))


@functools.lru_cache(maxsize=None)
def _make_edge_agg(Np, H, CH):
  rpt = Np // NS
  mesh = plsc.VectorSubcoreMesh(core_axis_name="c", subcore_axis_name="s")

  @functools.partial(
      pl.kernel,
      out_type=jax.ShapeDtypeStruct((NC, Np, H), jnp.float32),
      mesh=mesh,
      scratch_types=[
          pltpu.VMEM_SHARED((Np, H), jnp.float32),
          pltpu.VMEM((CH, K), jnp.int32),
          pltpu.VMEM((CH, K), jnp.int32),
          pltpu.VMEM((CH, K), jnp.float32),
          pltpu.VMEM((K, H), jnp.float32),
      ],
      compiler_params=pltpu.CompilerParams(use_tc_tiling_on_sc=False),
  )
  def edge_agg(table_h, src_h, dst_h, ew_h, z_h, out_h,
               acc, idx_s, idx_d, wts, rows):
    c = lax.axis_index("c")
    s = lax.axis_index("s")
    wid = c * NS + s
    pltpu.sync_copy(z_h.at[pl.ds(s * rpt, rpt)], acc.at[pl.ds(s * rpt, rpt)])
    pltpu.sync_copy(src_h.at[wid], idx_s)
    pltpu.sync_copy(dst_h.at[wid], idx_d)
    pltpu.sync_copy(ew_h.at[wid], wts)
    plsc.subcore_barrier()

    def chunk(j, carry):
      pltpu.sync_copy(table_h.at[idx_s.at[j]], rows)

      def group(g, carry2):
        wv = wts[j, pl.ds(g * 16, 16)]
        base = g * 16
        for i in range(16):
          e = base + i
          rows[e] = rows[e] * wv[i]
        return carry2

      lax.fori_loop(0, K // 16, group, 0)
      pltpu.sync_copy(rows, acc.at[idx_d.at[j]], add=True)
      return carry

    lax.fori_loop(0, CH, chunk, 0)
    plsc.subcore_barrier()
    pltpu.sync_copy(acc.at[pl.ds(s * rpt, rpt)],
                    out_h.at[c, pl.ds(s * rpt, rpt)])

  return edge_agg


def kernel(features, edge_index, edge_weight, self_weight, W1, b1, W2, b2):
  N, D = features.shape
  H1 = W1.shape[0]
  E = edge_index.shape[1]

  per_w = -(-E // NW)
  CH = -(-(-(-per_w // K)) // 8) * 8
  Ep = NW * CH * K
  pad = Ep - E
  Np = -(-N // (NS * 8)) * (NS * 8)

  src = edge_index[0]
  dst = edge_index[1]
  ew1 = edge_weight[:, 0] + 1.0
  src_r = jnp.concatenate([src, jnp.zeros((pad,), jnp.int32)]).reshape(NW, CH, K)
  dst_r = jnp.concatenate([dst, jnp.zeros((pad,), jnp.int32)]).reshape(NW, CH, K)
  ew_r = jnp.concatenate([ew1, jnp.zeros((pad,), jnp.float32)]).reshape(NW, CH, K)
  zeros_nh = jnp.zeros((Np, H1), jnp.float32)

  edge_agg = _make_edge_agg(Np, H1, CH)

  p1 = _matmul_tc(features, W1)
  agg1 = edge_agg(p1, src_r, dst_r, ew_r, zeros_nh)
  x = _combine_relu_tc(p1, self_weight, agg1[0, :N], agg1[1, :N], b1)
  agg2 = edge_agg(x, src_r, dst_r, ew_r, zeros_nh)
  out = _final_tc(x, self_weight, agg2[0, :N], agg2[1, :N], W2, b2)
  return out
